# baseline (speedup 1.0000x reference)
import functools

import jax
import jax.numpy as jnp
from jax import lax
from jax.experimental import pallas as pl
from jax.experimental.pallas import tpu as pltpu
from jax.experimental.pallas import tpu_sc as plsc

GRID = 128
CELLS = GRID * GRID
C = 196
CP = 208
HALF = CP // 2
NPLANES = 3
NSUB = 16
PCHUNK = 64
LN = 1024


def _idx_body(consts_ref, x_ref, y_ref, z_ref, i0_ref, i1_ref, i2_ref):
    s0 = consts_ref[0]
    o0 = consts_ref[1]
    s1 = consts_ref[2]
    o1 = consts_ref[3]
    s2 = consts_ref[4]
    o2 = consts_ref[5]

    def cell(u):
        g = ((u * 0.5 + 0.5) * (GRID - 1)).astype(jnp.int32)
        return jnp.clip(g, 0, GRID - 1)

    gx = cell(x_ref[0, 0] * s0 + o0)
    gy = cell(y_ref[0, 0] * s1 + o1)
    gz = cell(z_ref[0, 0] * s2 + o2)
    i0_ref[0] = gx * GRID + gy
    i1_ref[0] = gx * GRID + gz
    i2_ref[0] = gy * GRID + gz


def _w_body(rmod, consts_ref, a_ref, *refs):
    f_refs = refs[:8]
    w_ref = refs[8]
    cg = pl.program_id(1)
    alpha = jax.nn.sigmoid(a_ref[0, 0])
    nrow, ln = alpha.shape
    n2 = (lax.broadcasted_iota(jnp.int32, (nrow, ln), 0) * ln
          + lax.broadcasted_iota(jnp.int32, (nrow, ln), 1))
    s0 = consts_ref[0]
    o0 = consts_ref[1]
    s1 = consts_ref[2]
    o1 = consts_ref[3]
    s2 = consts_ref[4]
    o2 = consts_ref[5]
    for i in range(8):
        c = cg * 8 + i
        v = f_refs[i][0, 0]
        q = lax.rem(n2 + rmod * c, C)
        v = jnp.where(q == 0, v * s0 + o0,
            jnp.where(q == 1, v * s1 + o1,
            jnp.where(q == 2, v * s2 + o2, v)))
        v = jnp.where(c < C, v * alpha,
            jnp.where(c == C, alpha, jnp.zeros_like(v)))
        w_ref[0, i] = v


def _post_body(a0_ref, a1_ref, out_ref):
    m0 = a0_ref[0]
    m1 = a1_ref[0]
    w = jnp.maximum(m1[:, C - HALF:C - HALF + 1], 1e-6)
    t0 = jnp.transpose(m0 / w)
    t1 = jnp.transpose(m1[:, 0:96] / w)[0:C - HALF, :]
    out_ref[0, 0] = jnp.concatenate([t0, t1], axis=0)


def _make_sc_scatter(nbatch, npts):
    pts_per_sub = npts // NSUB
    nchunks = pts_per_sub // PCHUNK
    rows_per_sub = CELLS // NSUB
    npairs = nchunks // 2
    mesh = plsc.VectorSubcoreMesh(core_axis_name="c", subcore_axis_name="s")

    @functools.partial(
        pl.kernel,
        mesh=mesh,
        compiler_params=pltpu.CompilerParams(use_tc_tiling_on_sc=False),
        out_type=jax.ShapeDtypeStruct((2, nbatch, NPLANES, CELLS, HALF),
                                      jnp.float32),
        scratch_types=[
            pltpu.VMEM_SHARED((CELLS, HALF), jnp.float32),
            pltpu.VMEM((nchunks, PCHUNK), jnp.int32),
            pltpu.VMEM((PCHUNK, HALF), jnp.float32),
            pltpu.VMEM((PCHUNK, HALF), jnp.float32),
            pltpu.SemaphoreType.DMA,
            pltpu.SemaphoreType.DMA,
            pltpu.SemaphoreType.DMA,
        ],
    )
    def sc_scatter(wt_hbm, i0_hbm, i1_hbm, i2_hbm, z_hbm, out_hbm, acc,
                   idxm, rowsa, rowsb, sema, semb, semi):
        cid = lax.axis_index("c")
        sid = lax.axis_index("s")
        r0 = sid * rows_per_sub
        pbase = sid * pts_per_sub

        def plane_round(p, idx_hbm):
            def load(k, rows, sem, b):
                base = pbase + k * PCHUNK
                pltpu.async_copy(wt_hbm.at[cid, b, pl.ds(base, PCHUNK), :],
                                 rows, sem)

            def drain(rows, sem, b):
                pltpu.make_async_copy(
                    wt_hbm.at[cid, b, pl.ds(pbase, PCHUNK), :],
                    rows, sem).wait()

            def round_body(b, carry):
                pltpu.async_copy(
                    idx_hbm.at[b, pl.ds(sid * nchunks, nchunks), :],
                    idxm, semi)
                pltpu.sync_copy(z_hbm.at[pl.ds(r0, rows_per_sub), :],
                                acc.at[pl.ds(r0, rows_per_sub), :])
                pltpu.make_async_copy(
                    idx_hbm.at[b, pl.ds(sid * nchunks, nchunks), :],
                    idxm, semi).wait()
                plsc.subcore_barrier()
                load(0, rowsa, sema, b)

                def pair(i, carry2):
                    load(2 * i + 1, rowsb, semb, b)
                    drain(rowsa, sema, b)
                    pltpu.sync_copy(rowsa, acc.at[idxm.at[2 * i]], add=True)

                    @pl.when(i + 1 < npairs)
                    def _():
                        load(2 * i + 2, rowsa, sema, b)

                    drain(rowsb, semb, b)
                    pltpu.sync_copy(rowsb, acc.at[idxm.at[2 * i + 1]],
                                    add=True)
                    return carry2

                lax.fori_loop(0, npairs, pair, 0)
                plsc.subcore_barrier()
                pltpu.sync_copy(
                    acc.at[pl.ds(r0, rows_per_sub), :],
                    out_hbm.at[cid, b, p, pl.ds(r0, rows_per_sub), :])
                return carry

            lax.fori_loop(0, nbatch, round_body, 0)

        plane_round(0, i0_hbm)
        plane_round(1, i1_hbm)
        plane_round(2, i2_hbm)

    return sc_scatter


def kernel(GS_feats, scene_bounds):
    nbatch, npts, nchan = GS_feats.shape
    nrow = npts // LN
    sb = scene_bounds.astype(jnp.float32)
    s0 = 2.0 / (sb[1] - sb[0])
    o0 = -2.0 * sb[0] / (sb[1] - sb[0]) - 1.0
    s1 = 2.0 / (sb[3] - sb[2])
    o1 = -2.0 * sb[2] / (sb[3] - sb[2]) - 1.0
    s2 = 2.0 / (sb[5] - sb[4])
    o2 = -2.0 * sb[4] / (sb[5] - sb[4]) - 1.0
    consts = jnp.stack([s0, o0, s1, o1, s2, o2,
                        jnp.float32(0.0), jnp.float32(0.0)])

    p43 = jnp.transpose(GS_feats[:, :, 0:4], (0, 2, 1)).reshape(
        nbatch, 4, nrow, LN)
    flat3 = GS_feats.reshape(nbatch, (npts * nchan) // LN, LN)

    NB = 8
    nbr = nrow // NB
    i0, i1, i2 = pl.pallas_call(
        _idx_body,
        grid=(nbatch, nbr),
        in_specs=[
            pl.BlockSpec(memory_space=pltpu.SMEM),
            pl.BlockSpec((1, 1, NB, LN), lambda b, n: (b, 0, n, 0)),
            pl.BlockSpec((1, 1, NB, LN), lambda b, n: (b, 1, n, 0)),
            pl.BlockSpec((1, 1, NB, LN), lambda b, n: (b, 2, n, 0)),
        ],
        out_specs=[
            pl.BlockSpec((1, NB, LN), lambda b, n: (b, n, 0)),
            pl.BlockSpec((1, NB, LN), lambda b, n: (b, n, 0)),
            pl.BlockSpec((1, NB, LN), lambda b, n: (b, n, 0)),
        ],
        out_shape=[
            jax.ShapeDtypeStruct((nbatch, nrow, LN), jnp.int32),
            jax.ShapeDtypeStruct((nbatch, nrow, LN), jnp.int32),
            jax.ShapeDtypeStruct((nbatch, nrow, LN), jnp.int32),
        ],
    )(consts, p43, p43, p43)

    ngroups = CP // 8
    flat4 = flat3.reshape(nbatch, nchan, nrow, LN)
    wpad = pl.pallas_call(
        functools.partial(_w_body, npts % nchan),
        grid=(nbatch, ngroups),
        in_specs=[
            pl.BlockSpec(memory_space=pltpu.SMEM),
            pl.BlockSpec((1, 1, nrow, LN), lambda b, g: (b, 3, 0, 0)),
        ] + [
            pl.BlockSpec((1, 1, nrow, LN),
                         functools.partial(
                             lambda b, g, i=0:
                             (b, jnp.minimum(g * 8 + i, C - 1), 0, 0),
                             i=i))
            for i in range(8)
        ],
        out_specs=pl.BlockSpec((1, 8, nrow, LN), lambda b, g: (b, g, 0, 0)),
        out_shape=jax.ShapeDtypeStruct((nbatch, CP, nrow, LN), jnp.float32),
    )(consts, p43, *([flat4] * 8))

    wt = jnp.transpose(
        wpad.reshape(nbatch, 2, HALF, nrow, LN),
        (1, 0, 3, 4, 2)).reshape(2, nbatch, npts, HALF)
    zeros = jnp.zeros((CELLS, HALF), jnp.float32)
    i0f = i0.reshape(nbatch, npts // PCHUNK, PCHUNK)
    i1f = i1.reshape(nbatch, npts // PCHUNK, PCHUNK)
    i2f = i2.reshape(nbatch, npts // PCHUNK, PCHUNK)

    accs = _make_sc_scatter(nbatch, npts)(wt, i0f, i1f, i2f, zeros)

    Bc = 1024
    nr = nbatch * NPLANES
    a0 = accs[0].reshape(nr, CELLS, HALF)
    a1 = accs[1].reshape(nr, CELLS, HALF)
    out = pl.pallas_call(
        _post_body,
        grid=(nr, CELLS // Bc),
        in_specs=[
            pl.BlockSpec((1, Bc, HALF), lambda r, c2: (r, c2, 0)),
            pl.BlockSpec((1, Bc, HALF), lambda r, c2: (r, c2, 0)),
        ],
        out_specs=pl.BlockSpec((1, 1, C, Bc), lambda r, c2: (r, 0, 0, c2)),
        out_shape=jax.ShapeDtypeStruct((nr, 1, C, CELLS), jnp.float32),
    )(a0, a1)

    return out.reshape(nbatch, NPLANES, C, GRID, GRID)
